# Initial kernel scaffold; baseline (speedup 1.0000x reference)
import jax
import jax.numpy as jnp
from jax.experimental import pallas as pl


def kernel(x, fc_w, fc_b, rt_w, rt_b):
    raise NotImplementedError("write your pallas kernel here")


# speedup vs baseline: 2.9338x; 2.9338x over previous
import jax
import jax.numpy as jnp
from jax.experimental import pallas as pl
from jax.experimental.pallas import tpu as pltpu

_LAYER_DIMS = [(784, 512), (512, 256), (256, 128), (128, 64), (64, 32), (32, 10)]
_CFG = [(4, 2), (2, 2), (2, 2), (2, 2), (2, 2), (2, 4)]

_BB = 512


def _dot(a, b):
    return jax.lax.dot_general(
        a, b, (((1,), (1,)), ((), ())),
        preferred_element_type=jnp.float32,
        precision=jax.lax.Precision.HIGHEST,
    )


def _body(x_ref, *refs):
    w_refs = refs[0:6]
    b_refs = refs[6:12]
    rw_refs = refs[12:18]
    rb_refs = refs[18:24]
    o_ref = refs[24]
    bb = x_ref.shape[0]

    cur = x_ref[...]
    lane = jax.lax.broadcasted_iota(jnp.int32, cur.shape, 1)
    cur = jnp.where(lane == 0, 0.0, cur)

    for li in range(6):
        din, dout = _LAYER_DIMS[li]
        ig, og = _CFG[li]
        w = w_refs[li][...]
        b = b_refs[li][...]

        scores = _dot(cur, rw_refs[li][...]) + rb_refs[li][...]
        m = jnp.max(scores, axis=-1, keepdims=True)
        e = jnp.exp(scores - m)
        pw = e / jnp.sum(e, axis=-1, keepdims=True)

        wi = din // ig
        wo = [dout // og] * og
        wo[-1] = dout - (dout // og) * (og - 1)

        out = None
        for i in range(ig):
            a = _dot(cur[:, i * wi:(i + 1) * wi], w[:, i * wi:(i + 1) * wi]) + b
            g = jnp.concatenate(
                [jnp.broadcast_to(pw[:, i * og + j:i * og + j + 1], (bb, wo[j]))
                 for j in range(og)], axis=1)
            t = a * g
            out = t if out is None else out + t

        olane = jax.lax.broadcasted_iota(jnp.int32, out.shape, 1)
        out = jnp.where(olane == 0, 0.0, out)
        if li < 5:
            out = 0.5 * out * (1.0 + jax.lax.erf(out * 0.7071067811865476))
        cur = out

    o_ref[...] = cur


def kernel(x, fc_w, fc_b, rt_w, rt_b):
    batch = x.shape[0]
    bb = _BB if batch % _BB == 0 else batch
    b2 = [jnp.reshape(v, (1, -1)) for v in fc_b]
    rb2 = [jnp.reshape(v, (1, -1)) for v in rt_b]

    full = lambda arr: pl.BlockSpec(arr.shape, lambda i: (0, 0))
    in_specs = [pl.BlockSpec((bb, x.shape[1]), lambda i: (i, 0))]
    operands = [x]
    for group in (list(fc_w), b2, list(rt_w), rb2):
        for arr in group:
            in_specs.append(full(arr))
            operands.append(arr)

    return pl.pallas_call(
        _body,
        grid=(batch // bb,),
        in_specs=in_specs,
        out_specs=pl.BlockSpec((bb, 10), lambda i: (i, 0)),
        out_shape=jax.ShapeDtypeStruct((batch, 10), jnp.float32),
        compiler_params=pltpu.CompilerParams(
            dimension_semantics=("parallel",)),
    )(*operands)


# speedup vs baseline: 5.0828x; 1.7325x over previous
import jax
import jax.numpy as jnp
from jax.experimental import pallas as pl
from jax.experimental.pallas import tpu as pltpu

_LAYER_DIMS = [(784, 512), (512, 256), (256, 128), (128, 64), (64, 32), (32, 10)]
_CFG = [(4, 2), (2, 2), (2, 2), (2, 2), (2, 2), (2, 4)]

_BB = 512


def _dot(a, b):
    return jax.lax.dot_general(
        a, b, (((1,), (1,)), ((), ())),
        preferred_element_type=jnp.float32,
        precision=jax.lax.Precision.DEFAULT,
    )


def _body(x_ref, *refs):
    w_refs = refs[0:6]
    b_refs = refs[6:12]
    rw_refs = refs[12:18]
    rb_refs = refs[18:24]
    o_ref = refs[24]
    bb = x_ref.shape[0]

    cur = x_ref[...]
    lane = jax.lax.broadcasted_iota(jnp.int32, cur.shape, 1)
    cur = jnp.where(lane == 0, 0.0, cur)

    for li in range(6):
        din, dout = _LAYER_DIMS[li]
        ig, og = _CFG[li]
        w = w_refs[li][...]
        b = b_refs[li][...]

        scores = _dot(cur, rw_refs[li][...]) + rb_refs[li][...]
        m = jnp.max(scores, axis=-1, keepdims=True)
        e = jnp.exp(scores - m)
        pw = e / jnp.sum(e, axis=-1, keepdims=True)

        wi = din // ig
        wo = [dout // og] * og
        wo[-1] = dout - (dout // og) * (og - 1)

        out = None
        for i in range(ig):
            a = _dot(cur[:, i * wi:(i + 1) * wi], w[:, i * wi:(i + 1) * wi]) + b
            g = jnp.concatenate(
                [jnp.broadcast_to(pw[:, i * og + j:i * og + j + 1], (bb, wo[j]))
                 for j in range(og)], axis=1)
            t = a * g
            out = t if out is None else out + t

        olane = jax.lax.broadcasted_iota(jnp.int32, out.shape, 1)
        out = jnp.where(olane == 0, 0.0, out)
        if li < 5:
            out = 0.5 * out * (1.0 + jax.lax.erf(out * 0.7071067811865476))
        cur = out

    o_ref[...] = cur


def kernel(x, fc_w, fc_b, rt_w, rt_b):
    batch = x.shape[0]
    bb = _BB if batch % _BB == 0 else batch
    b2 = [jnp.reshape(v, (1, -1)) for v in fc_b]
    rb2 = [jnp.reshape(v, (1, -1)) for v in rt_b]

    full = lambda arr: pl.BlockSpec(arr.shape, lambda i: (0, 0))
    in_specs = [pl.BlockSpec((bb, x.shape[1]), lambda i: (i, 0))]
    operands = [x]
    for group in (list(fc_w), b2, list(rt_w), rb2):
        for arr in group:
            in_specs.append(full(arr))
            operands.append(arr)

    return pl.pallas_call(
        _body,
        grid=(batch // bb,),
        in_specs=in_specs,
        out_specs=pl.BlockSpec((bb, 10), lambda i: (i, 0)),
        out_shape=jax.ShapeDtypeStruct((batch, 10), jnp.float32),
        compiler_params=pltpu.CompilerParams(
            dimension_semantics=("parallel",)),
    )(*operands)


# speedup vs baseline: 5.5173x; 1.0855x over previous
import jax
import jax.numpy as jnp
from jax.experimental import pallas as pl
from jax.experimental.pallas import tpu as pltpu

_LAYER_DIMS = [(784, 512), (512, 256), (256, 128), (128, 64), (64, 32), (32, 10)]
_CFG = [(4, 2), (2, 2), (2, 2), (2, 2), (2, 2), (2, 4)]

_BB = 1024


def _dot(a, b):
    return jax.lax.dot_general(
        a, b, (((1,), (1,)), ((), ())),
        preferred_element_type=jnp.float32,
        precision=jax.lax.Precision.DEFAULT,
    )


def _body(x_ref, *refs):
    w_refs = refs[0:6]
    b_refs = refs[6:12]
    rw_refs = refs[12:18]
    rb_refs = refs[18:24]
    o_ref = refs[24]
    bb = x_ref.shape[0]

    cur = x_ref[...]
    lane = jax.lax.broadcasted_iota(jnp.int32, cur.shape, 1)
    cur = jnp.where(lane == 0, 0.0, cur)

    for li in range(6):
        din, dout = _LAYER_DIMS[li]
        ig, og = _CFG[li]
        w = w_refs[li][...]
        b = b_refs[li][...]

        scores = _dot(cur, rw_refs[li][...]) + rb_refs[li][...]
        m = jnp.max(scores, axis=-1, keepdims=True)
        e = jnp.exp(scores - m)
        pw = e / jnp.sum(e, axis=-1, keepdims=True)

        wi = din // ig
        wo = [dout // og] * og
        wo[-1] = dout - (dout // og) * (og - 1)

        out = None
        for i in range(ig):
            a = _dot(cur[:, i * wi:(i + 1) * wi], w[:, i * wi:(i + 1) * wi]) + b
            g = jnp.concatenate(
                [jnp.broadcast_to(pw[:, i * og + j:i * og + j + 1], (bb, wo[j]))
                 for j in range(og)], axis=1)
            t = a * g
            out = t if out is None else out + t

        olane = jax.lax.broadcasted_iota(jnp.int32, out.shape, 1)
        out = jnp.where(olane == 0, 0.0, out)
        if li < 5:
            out = 0.5 * out * (1.0 + jax.lax.erf(out * 0.7071067811865476))
        cur = out

    o_ref[...] = cur


def kernel(x, fc_w, fc_b, rt_w, rt_b):
    batch = x.shape[0]
    bb = _BB if batch % _BB == 0 else batch
    b2 = [jnp.reshape(v, (1, -1)) for v in fc_b]
    rb2 = [jnp.reshape(v, (1, -1)) for v in rt_b]

    full = lambda arr: pl.BlockSpec(arr.shape, lambda i: (0, 0))
    in_specs = [pl.BlockSpec((bb, x.shape[1]), lambda i: (i, 0))]
    operands = [x]
    for group in (list(fc_w), b2, list(rt_w), rb2):
        for arr in group:
            in_specs.append(full(arr))
            operands.append(arr)

    return pl.pallas_call(
        _body,
        grid=(batch // bb,),
        in_specs=in_specs,
        out_specs=pl.BlockSpec((bb, 10), lambda i: (i, 0)),
        out_shape=jax.ShapeDtypeStruct((batch, 10), jnp.float32),
        compiler_params=pltpu.CompilerParams(
            dimension_semantics=("parallel",)),
    )(*operands)


# speedup vs baseline: 5.5561x; 1.0070x over previous
import jax
import jax.numpy as jnp
from jax.experimental import pallas as pl
from jax.experimental.pallas import tpu as pltpu

_LAYER_DIMS = [(784, 512), (512, 256), (256, 128), (128, 64), (64, 32), (32, 10)]
_CFG = [(4, 2), (2, 2), (2, 2), (2, 2), (2, 2), (2, 4)]

_BB = 2048


def _dot(a, b):
    return jax.lax.dot_general(
        a, b, (((1,), (1,)), ((), ())),
        preferred_element_type=jnp.float32,
        precision=jax.lax.Precision.DEFAULT,
    )


def _body(x_ref, *refs):
    w_refs = refs[0:6]
    b_refs = refs[6:12]
    rw_refs = refs[12:18]
    rb_refs = refs[18:24]
    o_ref = refs[24]
    bb = x_ref.shape[0]

    cur = x_ref[...]
    lane = jax.lax.broadcasted_iota(jnp.int32, cur.shape, 1)
    cur = jnp.where(lane == 0, 0.0, cur)

    for li in range(6):
        din, dout = _LAYER_DIMS[li]
        ig, og = _CFG[li]
        w = w_refs[li][...]
        b = b_refs[li][...]

        scores = _dot(cur, rw_refs[li][...]) + rb_refs[li][...]
        m = jnp.max(scores, axis=-1, keepdims=True)
        e = jnp.exp(scores - m)
        pw = e / jnp.sum(e, axis=-1, keepdims=True)

        wi = din // ig
        wo = [dout // og] * og
        wo[-1] = dout - (dout // og) * (og - 1)

        out = None
        for i in range(ig):
            a = _dot(cur[:, i * wi:(i + 1) * wi], w[:, i * wi:(i + 1) * wi]) + b
            g = jnp.concatenate(
                [jnp.broadcast_to(pw[:, i * og + j:i * og + j + 1], (bb, wo[j]))
                 for j in range(og)], axis=1)
            t = a * g
            out = t if out is None else out + t

        olane = jax.lax.broadcasted_iota(jnp.int32, out.shape, 1)
        out = jnp.where(olane == 0, 0.0, out)
        if li < 5:
            out = 0.5 * out * (1.0 + jax.lax.erf(out * 0.7071067811865476))
        cur = out

    o_ref[...] = cur


def kernel(x, fc_w, fc_b, rt_w, rt_b):
    batch = x.shape[0]
    bb = _BB if batch % _BB == 0 else batch
    b2 = [jnp.reshape(v, (1, -1)) for v in fc_b]
    rb2 = [jnp.reshape(v, (1, -1)) for v in rt_b]

    full = lambda arr: pl.BlockSpec(arr.shape, lambda i: (0, 0))
    in_specs = [pl.BlockSpec((bb, x.shape[1]), lambda i: (i, 0))]
    operands = [x]
    for group in (list(fc_w), b2, list(rt_w), rb2):
        for arr in group:
            in_specs.append(full(arr))
            operands.append(arr)

    return pl.pallas_call(
        _body,
        grid=(batch // bb,),
        in_specs=in_specs,
        out_specs=pl.BlockSpec((bb, 10), lambda i: (i, 0)),
        out_shape=jax.ShapeDtypeStruct((batch, 10), jnp.float32),
        compiler_params=pltpu.CompilerParams(
            dimension_semantics=("parallel",)),
    )(*operands)
